# baseline (speedup 1.0000x reference)
import jax
import jax.numpy as jnp
from jax.experimental import pallas as pl

_K = 32
_L = 200
_NEG_INF = float("-inf")


def _topk_body(tw_ref, gate_ref, packed_ref, w_ref, tid_out_ref, attn_out_ref):
    tw = tw_ref[...].T
    gate = gate_ref[...].T
    packed = packed_ref[...].T
    l, c = tw.shape

    pos = jax.lax.broadcasted_iota(jnp.int32, (l, c), 0)
    s = jnp.sum(gate, axis=0, keepdims=True)
    need = s < _K
    keep = (pos >= 1) & (pos <= _K)
    unmask = (gate != 0) | (keep & need)
    twm = jnp.where(unmask, tw, _NEG_INF)

    kiota = jax.lax.broadcasted_iota(jnp.int32, (_K, c), 0)
    vals = jnp.zeros((_K, c), jnp.float32)
    keys = jnp.zeros((_K, c), jnp.int32)
    big = jnp.int32(1 << 30)
    for k in range(_K):
        m = jnp.max(twm, axis=0, keepdims=True)
        eq = twm == m
        minp = jnp.min(jnp.where(eq, packed, big), axis=0, keepdims=True)
        sel = packed == minp
        twm = jnp.where(sel, _NEG_INF, twm)
        vals = jnp.where(kiota == k, m, vals)
        keys = jnp.where(kiota == k, minp, keys)

    e = jnp.exp(vals - vals[0:1, :])
    w = e / jnp.sum(e, axis=0, keepdims=True)

    w_ref[...] = w
    tid_out_ref[...] = keys & 32767
    attn_out_ref[...] = (keys >> 15) & 1


def kernel(token_id, attn_mask, gate_mask, token_weight):
    b, l = token_weight.shape
    c = 512
    grid = (b // c,)

    pos = jnp.arange(l, dtype=jnp.int32)[None, :]
    packed = pos * 65536 + attn_mask * 32768 + token_id

    in_spec = pl.BlockSpec((c, l), lambda j: (j, 0))
    out_spec = pl.BlockSpec((_K, c), lambda j: (0, j))

    w_t, tid_o, attn_o = pl.pallas_call(
        _topk_body,
        grid=grid,
        in_specs=[in_spec, in_spec, in_spec],
        out_specs=[out_spec, out_spec, out_spec],
        out_shape=[
            jax.ShapeDtypeStruct((_K, b), jnp.float32),
            jax.ShapeDtypeStruct((_K, b), jnp.int32),
            jax.ShapeDtypeStruct((_K, b), jnp.int32),
        ],
    )(token_weight, gate_mask, packed)

    return (tid_o.T, attn_o.T, w_t.T)


# speedup vs baseline: 1.4244x; 1.4244x over previous
import jax
import jax.numpy as jnp
from jax.experimental import pallas as pl

_K = 32
_L = 200
_NEG_INF = float("-inf")


def _topk_body(tw_ref, gate_ref, packed_ref, w_ref, tid_out_ref, attn_out_ref):
    tw = tw_ref[...]
    gate = gate_ref[...]
    packed = packed_ref[...]
    l, c = tw.shape

    pos = jax.lax.broadcasted_iota(jnp.int32, (l, c), 0)
    s = jnp.sum(gate, axis=0, keepdims=True)
    need = s < _K
    keep = (pos >= 1) & (pos <= _K)
    unmask = (gate != 0) | (keep & need)
    twm = jnp.where(unmask, tw, _NEG_INF)

    kiota = jax.lax.broadcasted_iota(jnp.int32, (_K, c), 0)
    vals = jnp.zeros((_K, c), jnp.float32)
    keys = jnp.zeros((_K, c), jnp.int32)
    big = jnp.int32(1 << 30)
    for k in range(_K):
        m = jnp.max(twm, axis=0, keepdims=True)
        eq = twm == m
        minp = jnp.min(jnp.where(eq, packed, big), axis=0, keepdims=True)
        sel = packed == minp
        twm = jnp.where(sel, _NEG_INF, twm)
        vals = jnp.where(kiota == k, m, vals)
        keys = jnp.where(kiota == k, minp, keys)

    e = jnp.exp(vals - vals[0:1, :])
    w = e / jnp.sum(e, axis=0, keepdims=True)

    w_ref[...] = w
    tid_out_ref[...] = keys & 32767
    attn_out_ref[...] = (keys >> 15) & 1


def kernel(token_id, attn_mask, gate_mask, token_weight):
    b, l = token_weight.shape
    c = 512
    grid = (b // c,)

    pos = jnp.arange(l, dtype=jnp.int32)[None, :]
    packed = (pos * 65536 + attn_mask * 32768 + token_id).T
    tw_t = token_weight.T
    gate_t = gate_mask.T

    in_spec = pl.BlockSpec((l, c), lambda j: (0, j))
    out_spec = pl.BlockSpec((_K, c), lambda j: (0, j))

    w_t, tid_o, attn_o = pl.pallas_call(
        _topk_body,
        grid=grid,
        in_specs=[in_spec, in_spec, in_spec],
        out_specs=[out_spec, out_spec, out_spec],
        out_shape=[
            jax.ShapeDtypeStruct((_K, b), jnp.float32),
            jax.ShapeDtypeStruct((_K, b), jnp.int32),
            jax.ShapeDtypeStruct((_K, b), jnp.int32),
        ],
    )(tw_t, gate_t, packed)

    return (tid_o.T, attn_o.T, w_t.T)


# speedup vs baseline: 1.7056x; 1.1974x over previous
import jax
import jax.numpy as jnp
from jax.experimental import pallas as pl

_K = 32
_L = 200
_NEG_INF = float("-inf")


def _topk_body(tw_ref, tid_ref, gate_ref, attn_ref, w_ref, tid_out_ref, attn_out_ref):
    tw = tw_ref[...]
    tid = tid_ref[...]
    gate = gate_ref[...]
    attn = attn_ref[...]
    l, c = tw.shape

    pos = jax.lax.broadcasted_iota(jnp.int32, (l, c), 0)
    packed = pos * 65536 + attn * 32768 + tid
    s = jnp.sum(gate, axis=0, keepdims=True)
    need = s < _K
    keep = (pos >= 1) & (pos <= _K)
    unmask = (gate != 0) | (keep & need)
    twm = jnp.where(unmask, tw, _NEG_INF)

    kiota = jax.lax.broadcasted_iota(jnp.int32, (_K, c), 0)
    vals = jnp.zeros((_K, c), jnp.float32)
    keys = jnp.zeros((_K, c), jnp.int32)
    big = jnp.int32(1 << 30)
    for k in range(_K):
        m = jnp.max(twm, axis=0, keepdims=True)
        eq = twm == m
        minp = jnp.min(jnp.where(eq, packed, big), axis=0, keepdims=True)
        sel = packed == minp
        twm = jnp.where(sel, _NEG_INF, twm)
        vals = jnp.where(kiota == k, m, vals)
        keys = jnp.where(kiota == k, minp, keys)

    e = jnp.exp(vals - vals[0:1, :])
    w = e / jnp.sum(e, axis=0, keepdims=True)

    w_ref[...] = w
    tid_out_ref[...] = keys & 32767
    attn_out_ref[...] = (keys >> 15) & 1


def kernel(token_id, attn_mask, gate_mask, token_weight):
    b, l = token_weight.shape
    c = 1024
    grid = (b // c,)

    tw_t = token_weight.T
    tid_t = token_id.T
    gate_t = gate_mask.T
    attn_t = attn_mask.T

    in_spec = pl.BlockSpec((l, c), lambda j: (0, j))
    out_spec = pl.BlockSpec((_K, c), lambda j: (0, j))

    w_t, tid_o, attn_o = pl.pallas_call(
        _topk_body,
        grid=grid,
        in_specs=[in_spec, in_spec, in_spec, in_spec],
        out_specs=[out_spec, out_spec, out_spec],
        out_shape=[
            jax.ShapeDtypeStruct((_K, b), jnp.float32),
            jax.ShapeDtypeStruct((_K, b), jnp.int32),
            jax.ShapeDtypeStruct((_K, b), jnp.int32),
        ],
    )(tw_t, tid_t, gate_t, attn_t)

    return (tid_o.T, attn_o.T, w_t.T)


# speedup vs baseline: 1.7395x; 1.0199x over previous
import jax
import jax.numpy as jnp
from jax.experimental import pallas as pl

_K = 32
_L = 200
_NEG_INF = float("-inf")


def _topk_body(tw_ref, tid_ref, gate_ref, attn_ref, w_ref, tid_out_ref, attn_out_ref):
    tw = tw_ref[...]
    tid = tid_ref[...]
    gate = gate_ref[...]
    attn = attn_ref[...]
    l, c = tw.shape

    pos = jax.lax.broadcasted_iota(jnp.int32, (l, c), 0)
    packed = pos * 65536 + attn * 32768 + tid
    s = jnp.sum(gate, axis=0, keepdims=True)
    need = s < _K
    keep = (pos >= 1) & (pos <= _K)
    unmask = (gate != 0) | (keep & need)
    twm = jnp.where(unmask, tw, _NEG_INF)

    kiota = jax.lax.broadcasted_iota(jnp.int32, (_K, c), 0)
    vals = jnp.zeros((_K, c), jnp.float32)
    keys = jnp.zeros((_K, c), jnp.int32)
    big = jnp.int32(1 << 30)
    for k in range(_K):
        m = jnp.max(twm, axis=0, keepdims=True)
        eq = twm == m
        minp = jnp.min(jnp.where(eq, packed, big), axis=0, keepdims=True)
        sel = packed == minp
        twm = jnp.where(sel, _NEG_INF, twm)
        vals = jnp.where(kiota == k, m, vals)
        keys = jnp.where(kiota == k, minp, keys)

    e = jnp.exp(vals - vals[0:1, :])
    w = e / jnp.sum(e, axis=0, keepdims=True)

    w_ref[...] = w
    tid_out_ref[...] = keys & 32767
    attn_out_ref[...] = (keys >> 15) & 1


def kernel(token_id, attn_mask, gate_mask, token_weight):
    b, l = token_weight.shape
    c = 2048
    grid = (b // c,)

    tw_t = token_weight.T
    tid_t = token_id.T
    gate_t = gate_mask.T
    attn_t = attn_mask.T

    in_spec = pl.BlockSpec((l, c), lambda j: (0, j))
    out_spec = pl.BlockSpec((_K, c), lambda j: (0, j))

    w_t, tid_o, attn_o = pl.pallas_call(
        _topk_body,
        grid=grid,
        in_specs=[in_spec, in_spec, in_spec, in_spec],
        out_specs=[out_spec, out_spec, out_spec],
        out_shape=[
            jax.ShapeDtypeStruct((_K, b), jnp.float32),
            jax.ShapeDtypeStruct((_K, b), jnp.int32),
            jax.ShapeDtypeStruct((_K, b), jnp.int32),
        ],
    )(tw_t, tid_t, gate_t, attn_t)

    return (tid_o.T, attn_o.T, w_t.T)
